# baseline (speedup 1.0000x reference)
import jax
import jax.numpy as jnp
from jax.experimental import pallas as pl
from jax.experimental.pallas import tpu as pltpu

_GATE_THRESHOLD = 0.5
_GATE_TEMPERATURE = 1.0


def _gate_kernel(scale_ref, bias_ref, x_ref, w_ref, gh_ref, score_ref, mask_ref):
    x = x_ref[...].astype(jnp.bfloat16)
    w = w_ref[...]
    gh = jax.lax.dot_general(
        x, w, (((1,), (1,)), ((), ())), preferred_element_type=jnp.float32
    )
    gh_ref[...] = gh
    ghb = gh.astype(jnp.bfloat16)
    sq = ghb * ghb
    ones = jnp.ones((sq.shape[1], 128), dtype=jnp.bfloat16)
    sumsq = jax.lax.dot_general(
        sq, ones, (((1,), (0,)), ((), ())), preferred_element_type=jnp.float32
    )
    col = sumsq[:, 0:1]
    score = jnp.sqrt(col) * scale_ref[0, 0] - bias_ref[0, 0]
    keep = score >= (_GATE_THRESHOLD / _GATE_TEMPERATURE)
    score_ref[...] = jnp.where(keep, score, -jnp.inf)
    mask_ref[...] = keep.astype(jnp.float32)


def kernel(x, W_A, gate_scale, gate_bias):
    orig_shape = x.shape
    hidden = x.shape[-1]
    rank = W_A.shape[0]
    x_flat = x.reshape(-1, hidden)
    m = x_flat.shape[0]
    m_blk = 512 if m % 512 == 0 else m

    w_bf16 = W_A.astype(jnp.bfloat16)
    scale2 = gate_scale.reshape(1, 1)
    bias2 = gate_bias.reshape(1, 1)

    grid = m // m_blk
    gh, score_full, mask_f = pl.pallas_call(
        _gate_kernel,
        grid=(grid,),
        in_specs=[
            pl.BlockSpec(memory_space=pltpu.SMEM),
            pl.BlockSpec(memory_space=pltpu.SMEM),
            pl.BlockSpec((m_blk, hidden), lambda i: (i, 0)),
            pl.BlockSpec((rank, hidden), lambda i: (0, 0)),
        ],
        out_specs=[
            pl.BlockSpec((m_blk, rank), lambda i: (i, 0)),
            pl.BlockSpec((m_blk, 1), lambda i: (i, 0)),
            pl.BlockSpec((m_blk, 1), lambda i: (i, 0)),
        ],
        out_shape=[
            jax.ShapeDtypeStruct((m, rank), jnp.float32),
            jax.ShapeDtypeStruct((m, 1), jnp.float32),
            jax.ShapeDtypeStruct((m, 1), jnp.float32),
        ],
        compiler_params=pltpu.CompilerParams(
            dimension_semantics=("arbitrary",),
        ),
    )(scale2, bias2, x_flat, w_bf16)

    gate_mask_full = mask_f.astype(bool).reshape(orig_shape[:-1])
    gate_score_full = score_full.reshape(orig_shape[:-1])
    return (gate_mask_full, gate_score_full, gh)


# speedup vs baseline: 1.0457x; 1.0457x over previous
import jax
import jax.numpy as jnp
from jax.experimental import pallas as pl
from jax.experimental.pallas import tpu as pltpu

_GATE_THRESHOLD = 0.5
_GATE_TEMPERATURE = 1.0


def _gate_kernel(scale_ref, bias_ref, x_ref, w_ref, gh_ref, score_ref, mask_ref):
    x = x_ref[...]
    w = w_ref[...]
    gh = jax.lax.dot_general(
        x, w, (((1,), (1,)), ((), ())),
        precision=jax.lax.Precision.DEFAULT,
        preferred_element_type=jnp.float32,
    )
    gh_ref[...] = gh
    ghb = gh.astype(jnp.bfloat16)
    sq = ghb * ghb
    ones = jnp.ones((sq.shape[1], 128), dtype=jnp.bfloat16)
    sumsq = jax.lax.dot_general(
        sq, ones, (((1,), (0,)), ((), ())), preferred_element_type=jnp.float32
    )
    col = sumsq[:, 0:1]
    score = jnp.sqrt(col) * scale_ref[0, 0] - bias_ref[0, 0]
    keep = score >= (_GATE_THRESHOLD / _GATE_TEMPERATURE)
    score_ref[...] = jnp.where(keep, score, -jnp.inf)
    mask_ref[...] = keep.astype(jnp.float32)


def kernel(x, W_A, gate_scale, gate_bias):
    orig_shape = x.shape
    hidden = x.shape[-1]
    rank = W_A.shape[0]
    x_flat = x.reshape(-1, hidden)
    m = x_flat.shape[0]
    m_blk = 512 if m % 512 == 0 else m

    w_bf16 = W_A
    scale2 = gate_scale.reshape(1, 1)
    bias2 = gate_bias.reshape(1, 1)

    grid = m // m_blk
    gh, score_full, mask_f = pl.pallas_call(
        _gate_kernel,
        grid=(grid,),
        in_specs=[
            pl.BlockSpec(memory_space=pltpu.SMEM),
            pl.BlockSpec(memory_space=pltpu.SMEM),
            pl.BlockSpec((m_blk, hidden), lambda i: (i, 0)),
            pl.BlockSpec((rank, hidden), lambda i: (0, 0)),
        ],
        out_specs=[
            pl.BlockSpec((m_blk, rank), lambda i: (i, 0)),
            pl.BlockSpec((m_blk, 1), lambda i: (i, 0)),
            pl.BlockSpec((m_blk, 1), lambda i: (i, 0)),
        ],
        out_shape=[
            jax.ShapeDtypeStruct((m, rank), jnp.float32),
            jax.ShapeDtypeStruct((m, 1), jnp.float32),
            jax.ShapeDtypeStruct((m, 1), jnp.float32),
        ],
        compiler_params=pltpu.CompilerParams(
            dimension_semantics=("arbitrary",),
        ),
    )(scale2, bias2, x_flat, w_bf16)

    gate_mask_full = mask_f.astype(bool).reshape(orig_shape[:-1])
    gate_score_full = score_full.reshape(orig_shape[:-1])
    return (gate_mask_full, gate_score_full, gh)


# speedup vs baseline: 1.0674x; 1.0207x over previous
import jax
import jax.numpy as jnp
from jax.experimental import pallas as pl
from jax.experimental.pallas import tpu as pltpu

_GATE_THRESHOLD = 0.5
_GATE_TEMPERATURE = 1.0


def _gate_kernel(scale_ref, bias_ref, x_ref, w_ref, gh_ref, score_ref, mask_ref):
    x = x_ref[...]
    w = w_ref[...]
    gh = jax.lax.dot_general(
        x, w, (((1,), (1,)), ((), ())),
        precision=jax.lax.Precision.DEFAULT,
        preferred_element_type=jnp.float32,
    )
    gh_ref[...] = gh
    rank = gh.shape[1]
    n_chunks = max(1, rank // 256)
    chunk = rank // n_chunks
    ones = jnp.ones((chunk, 8), dtype=jnp.bfloat16)
    col = None
    for c in range(n_chunks):
        ghb = gh[:, c * chunk:(c + 1) * chunk].astype(jnp.bfloat16)
        sq = ghb * ghb
        part = jax.lax.dot_general(
            sq, ones, (((1,), (0,)), ((), ())),
            preferred_element_type=jnp.float32,
        )
        col = part if col is None else col + part
    col = col[:, 0:1]
    score = jnp.sqrt(col) * scale_ref[0, 0] - bias_ref[0, 0]
    keep = score >= (_GATE_THRESHOLD / _GATE_TEMPERATURE)
    score_ref[...] = jnp.where(keep, score, -jnp.inf)
    mask_ref[...] = keep.astype(jnp.float32)


def kernel(x, W_A, gate_scale, gate_bias):
    orig_shape = x.shape
    hidden = x.shape[-1]
    rank = W_A.shape[0]
    x_flat = x.reshape(-1, hidden)
    m = x_flat.shape[0]
    m_blk = 1024 if m % 1024 == 0 else m

    w_bf16 = W_A
    scale2 = gate_scale.reshape(1, 1)
    bias2 = gate_bias.reshape(1, 1)

    grid = m // m_blk
    gh, score_full, mask_f = pl.pallas_call(
        _gate_kernel,
        grid=(grid,),
        in_specs=[
            pl.BlockSpec(memory_space=pltpu.SMEM),
            pl.BlockSpec(memory_space=pltpu.SMEM),
            pl.BlockSpec((m_blk, hidden), lambda i: (i, 0)),
            pl.BlockSpec((rank, hidden), lambda i: (0, 0)),
        ],
        out_specs=[
            pl.BlockSpec((m_blk, rank), lambda i: (i, 0)),
            pl.BlockSpec((m_blk, 1), lambda i: (i, 0)),
            pl.BlockSpec((m_blk, 1), lambda i: (i, 0)),
        ],
        out_shape=[
            jax.ShapeDtypeStruct((m, rank), jnp.float32),
            jax.ShapeDtypeStruct((m, 1), jnp.float32),
            jax.ShapeDtypeStruct((m, 1), jnp.float32),
        ],
        compiler_params=pltpu.CompilerParams(
            dimension_semantics=("arbitrary",),
            vmem_limit_bytes=64 * 1024 * 1024,
        ),
    )(scale2, bias2, x_flat, w_bf16)

    gate_mask_full = mask_f.astype(bool).reshape(orig_shape[:-1])
    gate_score_full = score_full.reshape(orig_shape[:-1])
    return (gate_mask_full, gate_score_full, gh)
